# baseline (speedup 1.0000x reference)
import functools

import jax
import jax.numpy as jnp
from jax import lax
from jax.experimental import pallas as pl
from jax.experimental.pallas import tpu as pltpu
from jax.experimental.pallas import tpu_sc as plsc

N_CLASSES = 1000
N_CTX = 8
NAME_LEN = 77
OUT_LEN = N_CTX + NAME_LEN
CTX_DIM = 512

NW = 32
K_MAX = 32
WAVE = 8


def _class_copies(ctx_hbm, name_hbm, out_hbm, c, sems):
    c1 = pltpu.make_async_copy(
        ctx_hbm.at[c], out_hbm.at[c, pl.ds(0, N_CTX)], sems.at[0]
    )
    c2 = pltpu.make_async_copy(
        name_hbm.at[c], out_hbm.at[c, pl.ds(N_CTX, NAME_LEN)], sems.at[1]
    )
    return c1, c2


def kernel(ctx, name_embeds):
    mesh = plsc.VectorSubcoreMesh(core_axis_name="c", subcore_axis_name="s")

    @functools.partial(
        pl.kernel,
        mesh=mesh,
        out_type=jax.ShapeDtypeStruct((N_CLASSES, OUT_LEN, CTX_DIM), jnp.float32),
        scratch_types=[pltpu.SemaphoreType.DMA((2,))],
    )
    def _sc_concat(ctx_hbm, name_hbm, out_hbm, sems):
        wid = lax.axis_index("s") * 2 + lax.axis_index("c")

        def wave_body(w, _):
            def fire(k, _):
                c = wid + NW * (w * WAVE + k)

                @pl.when(c < N_CLASSES)
                def _():
                    c1, c2 = _class_copies(ctx_hbm, name_hbm, out_hbm, c, sems)
                    c1.start()
                    c2.start()

                return 0

            lax.fori_loop(0, WAVE, fire, 0)

            def drain(k, _):
                c = wid + NW * (w * WAVE + k)

                @pl.when(c < N_CLASSES)
                def _():
                    c1, c2 = _class_copies(ctx_hbm, name_hbm, out_hbm, c, sems)
                    c1.wait()
                    c2.wait()

                return 0

            lax.fori_loop(0, WAVE, drain, 0)
            return 0

        lax.fori_loop(0, K_MAX // WAVE, wave_body, 0)

    return _sc_concat(ctx, name_embeds)


# speedup vs baseline: 15.0183x; 15.0183x over previous
import functools

import jax
import jax.numpy as jnp
from jax import lax
from jax.experimental import pallas as pl
from jax.experimental.pallas import tpu as pltpu
from jax.experimental.pallas import tpu_sc as plsc

N_CLASSES = 1000
N_CTX = 8
NAME_LEN = 77
OUT_LEN = N_CTX + NAME_LEN
CTX_DIM = 512

NW = 32


def kernel(ctx, name_embeds):
    mesh = plsc.VectorSubcoreMesh(core_axis_name="c", subcore_axis_name="s")

    @functools.partial(
        pl.kernel,
        mesh=mesh,
        out_type=jax.ShapeDtypeStruct((N_CLASSES, OUT_LEN, CTX_DIM), jnp.float32),
        scratch_types=[
            pltpu.VMEM((2, OUT_LEN, CTX_DIM), jnp.float32),
            pltpu.SemaphoreType.DMA((2, 2)),
            pltpu.SemaphoreType.DMA((2,)),
        ],
    )
    def _sc_concat(ctx_hbm, name_hbm, out_hbm, buf, gsems, ssems):
        wid = lax.axis_index("s") * 2 + lax.axis_index("c")
        n = jnp.where(wid < 8, 32, 31)
        base = 31 * wid + jnp.minimum(wid, 8)

        def gathers(c, slot):
            g1 = pltpu.make_async_copy(
                ctx_hbm.at[c], buf.at[slot, pl.ds(0, N_CTX)], gsems.at[slot, 0]
            )
            g2 = pltpu.make_async_copy(
                name_hbm.at[c], buf.at[slot, pl.ds(N_CTX, NAME_LEN)], gsems.at[slot, 1]
            )
            return g1, g2

        def scatter(c, slot):
            return pltpu.make_async_copy(buf.at[slot], out_hbm.at[c], ssems.at[slot])

        def body(k, _):
            slot = k % 2

            @pl.when(k >= 2)
            def _():
                scatter(base + k - 2, slot).wait()

            @pl.when(k < n)
            def _():
                g1, g2 = gathers(base + k, slot)
                g1.start()
                g2.start()

            @pl.when(k >= 1)
            def _():
                g1, g2 = gathers(base + k - 1, 1 - slot)
                g1.wait()
                g2.wait()
                scatter(base + k - 1, 1 - slot).start()

            return 0

        lax.fori_loop(0, n + 1, body, 0)
        scatter(base + n - 1, (n - 1) % 2).wait()

    return _sc_concat(ctx, name_embeds)
